# Initial kernel scaffold; baseline (speedup 1.0000x reference)
import jax
import jax.numpy as jnp
from jax.experimental import pallas as pl


def kernel(adj, x, W0, a0, W1, a1, W2, a2, W3, a3, W_out, a_out):
    raise NotImplementedError("write your pallas kernel here")


# speedup vs baseline: 6.9919x; 6.9919x over previous
import functools

import jax
import jax.numpy as jnp
from jax import lax
from jax.experimental import pallas as pl
from jax.experimental.pallas import tpu as pltpu
from jax.experimental.pallas import tpu_sc as plsc

N = 10000
E = 320000
IN_DIM = 128
HID = 64
EMB = 128
NH = 4
ALPHA = 0.2
EPS = 1e-16

NC = 2
NS = 16
L = 16

RB = 400
GRID = N // RB
CH = 80


def _elu(v):
    return jnp.where(v > 0, v, jnp.exp(jnp.minimum(v, 0.0)) - 1.0)


_BCAST_DNUMS = lax.GatherDimensionNumbers(
    offset_dims=(), collapsed_slice_dims=(0,), start_index_map=(0,))


def _bcast(v16, lane):
    idx = jnp.broadcast_to(lane, (L,)).astype(jnp.int32)[:, None]
    return lax.gather(v16, idx, _BCAST_DNUMS, (1,),
                      mode=lax.GatherScatterMode.PROMISE_IN_BOUNDS)


def _tc1_body(x_ref, w_ref, a_ref, h_ref, st_ref):
    h = x_ref[...] @ w_ref[...]
    h_ref[0] = h[:, :128]
    h_ref[1] = h[:, 128:]
    st_ref[...] = h @ a_ref[...]


def _tc1(x, Wcat, A):
    return pl.pallas_call(
        _tc1_body,
        grid=(GRID,),
        in_specs=[
            pl.BlockSpec((RB, IN_DIM), lambda i: (i, 0)),
            pl.BlockSpec((IN_DIM, NH * HID), lambda i: (0, 0)),
            pl.BlockSpec((NH * HID, L), lambda i: (0, 0)),
        ],
        out_specs=[
            pl.BlockSpec((NC, RB, 2 * HID), lambda i: (0, i, 0)),
            pl.BlockSpec((RB, L), lambda i: (i, 0)),
        ],
        out_shape=[
            jax.ShapeDtypeStruct((NC, N, 2 * HID), jnp.float32),
            jax.ShapeDtypeStruct((N, L), jnp.float32),
        ],
    )(x, Wcat, A)


def _sc1a_body(src_hbm, dst_hbm, st_hbm, zr_hbm,
               rs_out, w4_out,
               src_v, dst_v, sg, tg, w4, rs_sh, st_sh, sem):
    c = lax.axis_index("c")
    s = lax.axis_index("s")

    @pl.when(s == 0)
    def _():
        pltpu.sync_copy(zr_hbm, rs_sh)
        pltpu.sync_copy(st_hbm, st_sh)

    plsc.subcore_barrier()

    shidx = jnp.minimum(lax.iota(jnp.int32, L) + 4, L - 1)[:, None]
    wid = s * NC + c
    per_w = E // (NC * NS)
    base0 = wid * per_w

    def chunk(ci, carry):
        base = base0 + ci * CH
        pltpu.sync_copy(src_hbm.at[pl.ds(base, CH)], src_v)
        pltpu.sync_copy(dst_hbm.at[pl.ds(base, CH)], dst_v)
        pltpu.async_copy(st_sh.at[src_v], sg, sem).wait()
        pltpu.async_copy(st_sh.at[dst_v], tg, sem).wait()

        def wcalc(g, carry2):
            kbase = g * L
            for j in range(L):
                k = kbase + j
                tsh = lax.gather(tg[k], shidx, _BCAST_DNUMS, (1,),
                                 mode=lax.GatherScatterMode.PROMISE_IN_BOUNDS)
                e = sg[k] + tsh
                le = jnp.where(e >= 0, e, ALPHA * e)
                w4[k] = jnp.exp(-le)
            return carry2

        lax.fori_loop(0, CH // L, wcalc, 0)

        pltpu.sync_copy(w4, rs_sh.at[src_v], add=True)
        pltpu.sync_copy(w4, w4_out.at[pl.ds(base, CH)])
        return carry

    lax.fori_loop(0, per_w // CH, chunk, 0)

    plsc.subcore_barrier()

    @pl.when(s == 0)
    def _():
        pltpu.sync_copy(rs_sh, rs_out.at[c])


def _sc1a(src, dst, ST, zr):
    mesh = plsc.VectorSubcoreMesh(core_axis_name="c", subcore_axis_name="s")
    kfn = pl.kernel(
        _sc1a_body,
        out_type=[
            jax.ShapeDtypeStruct((NC, N, L), jnp.float32),
            jax.ShapeDtypeStruct((E, L), jnp.float32),
        ],
        mesh=mesh,
        scratch_types=[
            pltpu.VMEM((CH,), jnp.int32),
            pltpu.VMEM((CH,), jnp.int32),
            pltpu.VMEM((CH, L), jnp.float32),
            pltpu.VMEM((CH, L), jnp.float32),
            pltpu.VMEM((CH, L), jnp.float32),
            pltpu.VMEM_SHARED((N, L), jnp.float32),
            pltpu.VMEM_SHARED((N, L), jnp.float32),
            pltpu.SemaphoreType.DMA,
        ],
    )
    return kfn(src, dst, ST, zr)


def _sc1b_body(src_hbm, dst_hbm, h0_hbm, h1_hbm, w4_hbm, zh_hbm,
               hp_out,
               src_v, dst_v, w4, rows, hp_sh, sem):
    c = lax.axis_index("c")
    s = lax.axis_index("s")

    @pl.when(s == 0)
    def _():
        pltpu.sync_copy(zh_hbm, hp_sh)

    plsc.subcore_barrier()

    c2 = 2 * c
    per_tile = E // NS
    base0 = s * per_tile

    def chunk(ci, carry):
        base = base0 + ci * CH
        pltpu.sync_copy(src_hbm.at[pl.ds(base, CH)], src_v)
        pltpu.sync_copy(dst_hbm.at[pl.ds(base, CH)], dst_v)
        pltpu.sync_copy(w4_hbm.at[pl.ds(base, CH)], w4)

        @pl.when(c == 0)
        def _():
            pltpu.async_copy(h0_hbm.at[dst_v], rows, sem).wait()

        @pl.when(c == 1)
        def _():
            pltpu.async_copy(h1_hbm.at[dst_v], rows, sem).wait()

        def scale(g, carry2):
            kbase = g * L
            for j in range(L):
                k = kbase + j
                wrow = w4[k]
                w0 = _bcast(wrow, c2)
                w1 = _bcast(wrow, c2 + 1)
                for f in range(4):
                    sl = pl.ds(f * L, L)
                    rows[k, sl] = rows[k, sl] * w0
                for f in range(4, 8):
                    sl = pl.ds(f * L, L)
                    rows[k, sl] = rows[k, sl] * w1
            return carry2

        lax.fori_loop(0, CH // L, scale, 0)

        pltpu.sync_copy(rows, hp_sh.at[src_v], add=True)
        return carry

    lax.fori_loop(0, per_tile // CH, chunk, 0)

    plsc.subcore_barrier()

    @pl.when(s == 0)
    def _():
        pltpu.sync_copy(hp_sh, hp_out.at[c])


def _sc1b(src, dst, H0, H1, W4, zh):
    mesh = plsc.VectorSubcoreMesh(core_axis_name="c", subcore_axis_name="s")
    kfn = pl.kernel(
        _sc1b_body,
        out_type=jax.ShapeDtypeStruct((NC, N, 2 * HID), jnp.float32),
        mesh=mesh,
        scratch_types=[
            pltpu.VMEM((CH,), jnp.int32),
            pltpu.VMEM((CH,), jnp.int32),
            pltpu.VMEM((CH, L), jnp.float32),
            pltpu.VMEM((CH, 2 * HID), jnp.float32),
            pltpu.VMEM_SHARED((N, 2 * HID), jnp.float32),
            pltpu.SemaphoreType.DMA,
        ],
    )
    return kfn(src, dst, H0, H1, W4, zh)


def _tc2_body(hp_ref, rs_ref, wo_ref, alt_ref, h2_ref, s2_ref, t2_ref):
    rs4 = rs_ref[0] + rs_ref[1]
    cols = []
    for i in range(NH):
        hpc = hp_ref[i // 2][:, (i % 2) * HID:(i % 2 + 1) * HID]
        cols.append(_elu(hpc / (rs4[:, i:i + 1] + EPS)))
    x2 = jnp.concatenate(cols, axis=1)
    h2 = x2 @ wo_ref[...]
    h2_ref[...] = h2
    st = h2 @ alt_ref[...]
    s2_ref[...] = st[:, 0:1]
    t2_ref[...] = st[:, 1:2]


def _tc2(hp, rs, W_out, ALT):
    return pl.pallas_call(
        _tc2_body,
        grid=(GRID,),
        in_specs=[
            pl.BlockSpec((NC, RB, 2 * HID), lambda i: (0, i, 0)),
            pl.BlockSpec((NC, RB, L), lambda i: (0, i, 0)),
            pl.BlockSpec((NH * HID, EMB), lambda i: (0, 0)),
            pl.BlockSpec((EMB, 2), lambda i: (0, 0)),
        ],
        out_specs=[
            pl.BlockSpec((RB, EMB), lambda i: (i, 0)),
            pl.BlockSpec((RB, 1), lambda i: (i, 0)),
            pl.BlockSpec((RB, 1), lambda i: (i, 0)),
        ],
        out_shape=[
            jax.ShapeDtypeStruct((N, EMB), jnp.float32),
            jax.ShapeDtypeStruct((N, 1), jnp.float32),
            jax.ShapeDtypeStruct((N, 1), jnp.float32),
        ],
    )(hp, rs, W_out, ALT)


def _sc2_body(src_hbm, dst_hbm, h2_hbm, s2_hbm, t2_hbm, zh_hbm, zr_hbm,
              hp_out, rs_out, att_out,
              src_v, dst_v, sg, tg, wv, rows, hp_sh, rs_sh, s_sh, t_sh, sem):
    c = lax.axis_index("c")
    s = lax.axis_index("s")

    @pl.when(s == 0)
    def _():
        pltpu.sync_copy(zh_hbm, hp_sh)
        pltpu.sync_copy(zr_hbm, rs_sh)
        pltpu.sync_copy(s2_hbm, s_sh)
        pltpu.sync_copy(t2_hbm, t_sh)

    plsc.subcore_barrier()

    wid = s * NC + c
    per_tile = E // (NC * NS)
    base0 = wid * per_tile

    def chunk(ci, carry):
        base = base0 + ci * CH
        pltpu.sync_copy(src_hbm.at[pl.ds(base, CH)], src_v)
        pltpu.sync_copy(dst_hbm.at[pl.ds(base, CH)], dst_v)
        pltpu.async_copy(s_sh.at[src_v], sg, sem).wait()
        pltpu.async_copy(t_sh.at[dst_v], tg, sem).wait()

        def wstep(i, carry2):
            sl = pl.ds(i * L, L)
            e = sg[sl] + tg[sl]
            le = jnp.where(e >= 0, e, ALPHA * e)
            wv[sl] = jnp.exp(-le)
            return carry2

        lax.fori_loop(0, CH // L, wstep, 0)

        pltpu.sync_copy(wv, rs_sh.at[src_v], add=True)
        pltpu.sync_copy(wv, att_out.at[pl.ds(base, CH)])

        pltpu.async_copy(h2_hbm.at[dst_v], rows, sem).wait()

        def scale(g, carry2):
            kbase = g * L
            wgrp = wv[pl.ds(kbase, L)]
            for j in range(L):
                k = kbase + j
                w0 = _bcast(wgrp, j)
                for f in range(8):
                    sl = pl.ds(f * L, L)
                    rows[k, sl] = rows[k, sl] * w0
            return carry2

        lax.fori_loop(0, CH // L, scale, 0)

        pltpu.sync_copy(rows, hp_sh.at[src_v], add=True)
        return carry

    lax.fori_loop(0, per_tile // CH, chunk, 0)

    plsc.subcore_barrier()

    @pl.when(s == 0)
    def _():
        pltpu.sync_copy(hp_sh, hp_out.at[c])
        pltpu.sync_copy(rs_sh, rs_out.at[c])


def _sc2(src, dst, h2, s2, t2, zh, zr1):
    mesh = plsc.VectorSubcoreMesh(core_axis_name="c", subcore_axis_name="s")
    kfn = pl.kernel(
        _sc2_body,
        out_type=[
            jax.ShapeDtypeStruct((NC, N, EMB), jnp.float32),
            jax.ShapeDtypeStruct((NC, N), jnp.float32),
            jax.ShapeDtypeStruct((E,), jnp.float32),
        ],
        mesh=mesh,
        scratch_types=[
            pltpu.VMEM((CH,), jnp.int32),
            pltpu.VMEM((CH,), jnp.int32),
            pltpu.VMEM((CH,), jnp.float32),
            pltpu.VMEM((CH,), jnp.float32),
            pltpu.VMEM((CH,), jnp.float32),
            pltpu.VMEM((CH, EMB), jnp.float32),
            pltpu.VMEM_SHARED((N, EMB), jnp.float32),
            pltpu.VMEM_SHARED((N,), jnp.float32),
            pltpu.VMEM_SHARED((N,), jnp.float32),
            pltpu.VMEM_SHARED((N,), jnp.float32),
            pltpu.SemaphoreType.DMA,
        ],
    )
    return kfn(src, dst, h2, s2, t2, zh, zr1)


def _tc3_body(hp_ref, rs_ref, out_ref):
    acc = hp_ref[0] + hp_ref[1]
    rsum = rs_ref[0] + rs_ref[1] + EPS
    out_ref[...] = _elu(acc / rsum)


def _tc3(hp2, rs2):
    return pl.pallas_call(
        _tc3_body,
        grid=(GRID,),
        in_specs=[
            pl.BlockSpec((NC, RB, EMB), lambda i: (0, i, 0)),
            pl.BlockSpec((NC, RB, 1), lambda i: (0, i, 0)),
        ],
        out_specs=pl.BlockSpec((RB, EMB), lambda i: (i, 0)),
        out_shape=jax.ShapeDtypeStruct((N, EMB), jnp.float32),
    )(hp2, rs2)


def kernel(adj, x, W0, a0, W1, a1, W2, a2, W3, a3, W_out, a_out):
    adj32 = adj.astype(jnp.int32)
    src = adj32[0]
    dst = adj32[1]

    Wcat = jnp.concatenate([W0, W1, W2, W3], axis=1)
    A = jnp.zeros((NH * HID, L), jnp.float32)
    for i, a in enumerate([a0, a1, a2, a3]):
        A = A.at[i * HID:(i + 1) * HID, i].set(a[0, :HID])
        A = A.at[i * HID:(i + 1) * HID, 4 + i].set(a[0, HID:])
    ALT = jnp.concatenate([a_out[:, :EMB].T, a_out[:, EMB:].T], axis=1)

    zh = jnp.zeros((N, 2 * HID), jnp.float32)
    zr = jnp.zeros((N, L), jnp.float32)
    zr1 = jnp.zeros((N,), jnp.float32)

    H, ST = _tc1(x, Wcat, A)
    rs, W4 = _sc1a(src, dst, ST, zr)
    hp = _sc1b(src, dst, H[0], H[1], W4, zh)
    h2, s2, t2 = _tc2(hp, rs, W_out, ALT)
    hp2, rs2, att = _sc2(src, dst, h2, s2.reshape(N), t2.reshape(N), zh, zr1)
    out = _tc3(hp2, rs2.reshape(NC, N, 1))
    return out, adj, att


# speedup vs baseline: 10.1474x; 1.4513x over previous
import functools

import jax
import jax.numpy as jnp
from jax import lax
from jax.experimental import pallas as pl
from jax.experimental.pallas import tpu as pltpu
from jax.experimental.pallas import tpu_sc as plsc

N = 10000
E = 320000
IN_DIM = 128
HID = 64
EMB = 128
NH = 4
ALPHA = 0.2
EPS = 1e-16

NC = 2
NS = 16
L = 16

RB = 400
GRID = N // RB
CH = 80
NR0 = 624
NR_LAST = N - (NS - 1) * NR0


def _elu(v):
    return jnp.where(v > 0, v, jnp.exp(jnp.minimum(v, 0.0)) - 1.0)


_BCAST_DNUMS = lax.GatherDimensionNumbers(
    offset_dims=(), collapsed_slice_dims=(0,), start_index_map=(0,))


def _bcast(v16, lane):
    idx = jnp.broadcast_to(lane, (L,)).astype(jnp.int32)[:, None]
    return lax.gather(v16, idx, _BCAST_DNUMS, (1,),
                      mode=lax.GatherScatterMode.PROMISE_IN_BOUNDS)


def _tc1_body(x_ref, w_ref, a_ref, h_ref, st_ref):
    h = x_ref[...] @ w_ref[...]
    h_ref[0] = h[:, :128]
    h_ref[1] = h[:, 128:]
    st_ref[...] = h @ a_ref[...]


def _tc1(x, Wcat, A):
    return pl.pallas_call(
        _tc1_body,
        grid=(GRID,),
        in_specs=[
            pl.BlockSpec((RB, IN_DIM), lambda i: (i, 0)),
            pl.BlockSpec((IN_DIM, NH * HID), lambda i: (0, 0)),
            pl.BlockSpec((NH * HID, L), lambda i: (0, 0)),
        ],
        out_specs=[
            pl.BlockSpec((NC, RB, 2 * HID), lambda i: (0, i, 0)),
            pl.BlockSpec((RB, L), lambda i: (i, 0)),
        ],
        out_shape=[
            jax.ShapeDtypeStruct((NC, N, 2 * HID), jnp.float32),
            jax.ShapeDtypeStruct((N, L), jnp.float32),
        ],
    )(x, Wcat, A)


def _sc1_body(src_hbm, dst_hbm, stf_hbm, h_hbm, zh_hbm, zr_hbm,
              rs_out, hp_out,
              src_v, dst_v, is0, is1, it0, it1, s0g, s1g, t0g, t1g,
              w0v, w1v, rows, rs0_sh, rs1_sh, st_sh, hp_sh,
              sem_r, sem_0, sem_1, sem_2, sem_3):
    c = lax.axis_index("c")
    s = lax.axis_index("s")
    c2 = 2 * c

    @pl.when(s == 1)
    def _():
        pltpu.sync_copy(stf_hbm, st_sh)

    @pl.when(s < NS - 1)
    def _():
        sl_hp = pl.ds(s * NR0, NR0)
        pltpu.sync_copy(zh_hbm.at[sl_hp], hp_sh.at[sl_hp])

    @pl.when(s == NS - 1)
    def _():
        sl_hp = pl.ds((NS - 1) * NR0, NR_LAST)
        pltpu.sync_copy(zh_hbm.at[sl_hp], hp_sh.at[sl_hp])

    @pl.when(s == 0)
    def _():
        pltpu.sync_copy(zr_hbm, rs0_sh)
        pltpu.sync_copy(zr_hbm, rs1_sh)

    plsc.subcore_barrier()

    per_tile = E // NS
    base0 = s * per_tile
    G = CH // L

    def chunk(ci, carry):
        base = base0 + ci * CH
        pltpu.sync_copy(src_hbm.at[pl.ds(base, CH)], src_v)
        pltpu.sync_copy(dst_hbm.at[pl.ds(base, CH)], dst_v)

        cp_r = pltpu.async_copy(h_hbm.at[c].at[dst_v], rows, sem_r)

        def idxcalc(g, carry2):
            sl = pl.ds(g * L, L)
            s16 = src_v[sl] * L
            d16 = dst_v[sl] * L
            is0[sl] = s16 + c2
            is1[sl] = s16 + (c2 + 1)
            it0[sl] = d16 + (c2 + 4)
            it1[sl] = d16 + (c2 + 5)
            return carry2

        lax.fori_loop(0, G, idxcalc, 0)

        cp0 = pltpu.async_copy(st_sh.at[is0], s0g, sem_0)
        cp1 = pltpu.async_copy(st_sh.at[is1], s1g, sem_1)
        cp2 = pltpu.async_copy(st_sh.at[it0], t0g, sem_2)
        cp3 = pltpu.async_copy(st_sh.at[it1], t1g, sem_3)
        cp0.wait()
        cp1.wait()
        cp2.wait()
        cp3.wait()

        def wcalc(g, carry2):
            sl = pl.ds(g * L, L)
            e0 = s0g[sl] + t0g[sl]
            w0v[sl] = jnp.exp(-jnp.where(e0 >= 0, e0, ALPHA * e0))
            e1 = s1g[sl] + t1g[sl]
            w1v[sl] = jnp.exp(-jnp.where(e1 >= 0, e1, ALPHA * e1))
            return carry2

        lax.fori_loop(0, G, wcalc, 0)

        pltpu.sync_copy(w0v, rs0_sh.at[src_v], add=True)
        pltpu.sync_copy(w1v, rs1_sh.at[src_v], add=True)

        cp_r.wait()

        def scale(g, carry2):
            kbase = g * L
            w0g = w0v[pl.ds(kbase, L)]
            w1g = w1v[pl.ds(kbase, L)]
            for j in range(L):
                k = kbase + j
                b0 = _bcast(w0g, j)
                b1 = _bcast(w1g, j)
                for f in range(4):
                    sl = pl.ds(f * L, L)
                    rows[k, sl] = rows[k, sl] * b0
                for f in range(4, 8):
                    sl = pl.ds(f * L, L)
                    rows[k, sl] = rows[k, sl] * b1
            return carry2

        lax.fori_loop(0, G, scale, 0)

        pltpu.sync_copy(rows, hp_sh.at[src_v], add=True)
        return carry

    lax.fori_loop(0, per_tile // CH, chunk, 0)

    plsc.subcore_barrier()

    @pl.when(s < NS - 1)
    def _():
        sl_hp = pl.ds(s * NR0, NR0)
        pltpu.sync_copy(hp_sh.at[sl_hp], hp_out.at[c].at[sl_hp])

    @pl.when(s == NS - 1)
    def _():
        sl_hp = pl.ds((NS - 1) * NR0, NR_LAST)
        pltpu.sync_copy(hp_sh.at[sl_hp], hp_out.at[c].at[sl_hp])

    @pl.when(s == 0)
    def _():
        pltpu.sync_copy(rs0_sh, rs_out.at[c2])
        pltpu.sync_copy(rs1_sh, rs_out.at[c2 + 1])


def _sc1(src, dst, STf, H, zh, zr1):
    mesh = plsc.VectorSubcoreMesh(core_axis_name="c", subcore_axis_name="s")
    kfn = pl.kernel(
        _sc1_body,
        out_type=[
            jax.ShapeDtypeStruct((NH, N), jnp.float32),
            jax.ShapeDtypeStruct((NC, N, 2 * HID), jnp.float32),
        ],
        mesh=mesh,
        scratch_types=[
            pltpu.VMEM((CH,), jnp.int32),
            pltpu.VMEM((CH,), jnp.int32),
            pltpu.VMEM((CH,), jnp.int32),
            pltpu.VMEM((CH,), jnp.int32),
            pltpu.VMEM((CH,), jnp.int32),
            pltpu.VMEM((CH,), jnp.int32),
            pltpu.VMEM((CH,), jnp.float32),
            pltpu.VMEM((CH,), jnp.float32),
            pltpu.VMEM((CH,), jnp.float32),
            pltpu.VMEM((CH,), jnp.float32),
            pltpu.VMEM((CH,), jnp.float32),
            pltpu.VMEM((CH,), jnp.float32),
            pltpu.VMEM((CH, 2 * HID), jnp.float32),
            pltpu.VMEM_SHARED((N,), jnp.float32),
            pltpu.VMEM_SHARED((N,), jnp.float32),
            pltpu.VMEM_SHARED((N * L,), jnp.float32),
            pltpu.VMEM_SHARED((N, 2 * HID), jnp.float32),
            pltpu.SemaphoreType.DMA,
            pltpu.SemaphoreType.DMA,
            pltpu.SemaphoreType.DMA,
            pltpu.SemaphoreType.DMA,
            pltpu.SemaphoreType.DMA,
        ],
    )
    return kfn(src, dst, STf, H, zh, zr1)


def _tc2_body(hp_ref, r0_ref, r1_ref, r2_ref, r3_ref, wo_ref, alt_ref,
              h2_ref, s2_ref, t2_ref):
    rs = [r0_ref, r1_ref, r2_ref, r3_ref]
    cols = []
    for i in range(NH):
        hpc = hp_ref[i // 2][:, (i % 2) * HID:(i % 2 + 1) * HID]
        cols.append(_elu(hpc / (rs[i][...] + EPS)))
    x2 = jnp.concatenate(cols, axis=1)
    h2 = x2 @ wo_ref[...]
    h2_ref[...] = h2
    st = h2 @ alt_ref[...]
    s2_ref[...] = st[:, 0:1]
    t2_ref[...] = st[:, 1:2]


def _tc2(hp, r0, r1, r2, r3, W_out, ALT):
    return pl.pallas_call(
        _tc2_body,
        grid=(GRID,),
        in_specs=[
            pl.BlockSpec((NC, RB, 2 * HID), lambda i: (0, i, 0)),
            pl.BlockSpec((RB, 1), lambda i: (i, 0)),
            pl.BlockSpec((RB, 1), lambda i: (i, 0)),
            pl.BlockSpec((RB, 1), lambda i: (i, 0)),
            pl.BlockSpec((RB, 1), lambda i: (i, 0)),
            pl.BlockSpec((NH * HID, EMB), lambda i: (0, 0)),
            pl.BlockSpec((EMB, 2), lambda i: (0, 0)),
        ],
        out_specs=[
            pl.BlockSpec((RB, EMB), lambda i: (i, 0)),
            pl.BlockSpec((RB, 1), lambda i: (i, 0)),
            pl.BlockSpec((RB, 1), lambda i: (i, 0)),
        ],
        out_shape=[
            jax.ShapeDtypeStruct((N, EMB), jnp.float32),
            jax.ShapeDtypeStruct((N, 1), jnp.float32),
            jax.ShapeDtypeStruct((N, 1), jnp.float32),
        ],
    )(hp, r0, r1, r2, r3, W_out, ALT)


def _sc2_body(src_hbm, dst_hbm, h2_hbm, s2_hbm, t2_hbm, zh_hbm, zr_hbm,
              hp_out, rs_out, att_out,
              src_v, dst_v, sg, tg, wv, rows, hp_sh, rs_sh, s_sh, t_sh,
              sem, sem_r):
    c = lax.axis_index("c")
    s = lax.axis_index("s")

    @pl.when(s < NS - 1)
    def _():
        sl_hp = pl.ds(s * NR0, NR0)
        pltpu.sync_copy(zh_hbm.at[sl_hp], hp_sh.at[sl_hp])

    @pl.when(s == NS - 1)
    def _():
        sl_hp = pl.ds((NS - 1) * NR0, NR_LAST)
        pltpu.sync_copy(zh_hbm.at[sl_hp], hp_sh.at[sl_hp])

    @pl.when(s == 0)
    def _():
        pltpu.sync_copy(zr_hbm, rs_sh)
        pltpu.sync_copy(s2_hbm, s_sh)
        pltpu.sync_copy(t2_hbm, t_sh)

    plsc.subcore_barrier()

    wid = s * NC + c
    per_tile = E // (NC * NS)
    base0 = wid * per_tile

    def chunk(ci, carry):
        base = base0 + ci * CH
        pltpu.sync_copy(src_hbm.at[pl.ds(base, CH)], src_v)
        pltpu.sync_copy(dst_hbm.at[pl.ds(base, CH)], dst_v)

        cp_r = pltpu.async_copy(h2_hbm.at[dst_v], rows, sem_r)

        pltpu.async_copy(s_sh.at[src_v], sg, sem).wait()
        pltpu.async_copy(t_sh.at[dst_v], tg, sem).wait()

        def wstep(i, carry2):
            sl = pl.ds(i * L, L)
            e = sg[sl] + tg[sl]
            le = jnp.where(e >= 0, e, ALPHA * e)
            wv[sl] = jnp.exp(-le)
            return carry2

        lax.fori_loop(0, CH // L, wstep, 0)

        pltpu.sync_copy(wv, rs_sh.at[src_v], add=True)
        pltpu.sync_copy(wv, att_out.at[pl.ds(base, CH)])

        cp_r.wait()

        def scale(g, carry2):
            kbase = g * L
            wgrp = wv[pl.ds(kbase, L)]
            for j in range(L):
                k = kbase + j
                w0 = _bcast(wgrp, j)
                for f in range(8):
                    sl = pl.ds(f * L, L)
                    rows[k, sl] = rows[k, sl] * w0
            return carry2

        lax.fori_loop(0, CH // L, scale, 0)

        pltpu.sync_copy(rows, hp_sh.at[src_v], add=True)
        return carry

    lax.fori_loop(0, per_tile // CH, chunk, 0)

    plsc.subcore_barrier()

    @pl.when(s < NS - 1)
    def _():
        sl_hp = pl.ds(s * NR0, NR0)
        pltpu.sync_copy(hp_sh.at[sl_hp], hp_out.at[c].at[sl_hp])

    @pl.when(s == NS - 1)
    def _():
        sl_hp = pl.ds((NS - 1) * NR0, NR_LAST)
        pltpu.sync_copy(hp_sh.at[sl_hp], hp_out.at[c].at[sl_hp])

    @pl.when(s == 0)
    def _():
        pltpu.sync_copy(rs_sh, rs_out.at[c])


def _sc2(src, dst, h2, s2, t2, zh, zr1):
    mesh = plsc.VectorSubcoreMesh(core_axis_name="c", subcore_axis_name="s")
    kfn = pl.kernel(
        _sc2_body,
        out_type=[
            jax.ShapeDtypeStruct((NC, N, EMB), jnp.float32),
            jax.ShapeDtypeStruct((NC, N), jnp.float32),
            jax.ShapeDtypeStruct((E,), jnp.float32),
        ],
        mesh=mesh,
        scratch_types=[
            pltpu.VMEM((CH,), jnp.int32),
            pltpu.VMEM((CH,), jnp.int32),
            pltpu.VMEM((CH,), jnp.float32),
            pltpu.VMEM((CH,), jnp.float32),
            pltpu.VMEM((CH,), jnp.float32),
            pltpu.VMEM((CH, EMB), jnp.float32),
            pltpu.VMEM_SHARED((N, EMB), jnp.float32),
            pltpu.VMEM_SHARED((N,), jnp.float32),
            pltpu.VMEM_SHARED((N,), jnp.float32),
            pltpu.VMEM_SHARED((N,), jnp.float32),
            pltpu.SemaphoreType.DMA,
            pltpu.SemaphoreType.DMA,
        ],
    )
    return kfn(src, dst, h2, s2, t2, zh, zr1)


def _tc3_body(hp_ref, rs_ref, out_ref):
    acc = hp_ref[0] + hp_ref[1]
    rsum = rs_ref[0] + rs_ref[1] + EPS
    out_ref[...] = _elu(acc / rsum)


def _tc3(hp2, rs2):
    return pl.pallas_call(
        _tc3_body,
        grid=(GRID,),
        in_specs=[
            pl.BlockSpec((NC, RB, EMB), lambda i: (0, i, 0)),
            pl.BlockSpec((NC, RB, 1), lambda i: (0, i, 0)),
        ],
        out_specs=pl.BlockSpec((RB, EMB), lambda i: (i, 0)),
        out_shape=jax.ShapeDtypeStruct((N, EMB), jnp.float32),
    )(hp2, rs2)


def kernel(adj, x, W0, a0, W1, a1, W2, a2, W3, a3, W_out, a_out):
    adj32 = adj.astype(jnp.int32)
    src = adj32[0]
    dst = adj32[1]

    Wcat = jnp.concatenate([W0, W1, W2, W3], axis=1)
    A = jnp.zeros((NH * HID, L), jnp.float32)
    for i, a in enumerate([a0, a1, a2, a3]):
        A = A.at[i * HID:(i + 1) * HID, i].set(a[0, :HID])
        A = A.at[i * HID:(i + 1) * HID, 4 + i].set(a[0, HID:])
    ALT = jnp.concatenate([a_out[:, :EMB].T, a_out[:, EMB:].T], axis=1)

    zh = jnp.zeros((N, 2 * HID), jnp.float32)
    zr1 = jnp.zeros((N,), jnp.float32)

    H, ST = _tc1(x, Wcat, A)
    rs, hp = _sc1(src, dst, ST.reshape(N * L), H, zh, zr1)
    h2, s2, t2 = _tc2(hp, rs[0].reshape(N, 1), rs[1].reshape(N, 1),
                      rs[2].reshape(N, 1), rs[3].reshape(N, 1), W_out, ALT)
    hp2, rs2, att = _sc2(src, dst, h2, s2.reshape(N), t2.reshape(N), zh, zr1)
    out = _tc3(hp2, rs2.reshape(NC, N, 1))
    return out, adj, att


# speedup vs baseline: 13.8282x; 1.3627x over previous
import functools

import jax
import jax.numpy as jnp
from jax import lax
from jax.experimental import pallas as pl
from jax.experimental.pallas import tpu as pltpu
from jax.experimental.pallas import tpu_sc as plsc

N = 10000
E = 320000
IN_DIM = 128
HID = 64
EMB = 128
NH = 4
ALPHA = 0.2
EPS = 1e-16

NC = 2
NS = 16
L = 16

RB = 400
GRID = N // RB
CH = 80
NR0 = 624
NR_LAST = N - (NS - 1) * NR0


def _elu(v):
    return jnp.where(v > 0, v, jnp.exp(jnp.minimum(v, 0.0)) - 1.0)


_BCAST_DNUMS = lax.GatherDimensionNumbers(
    offset_dims=(), collapsed_slice_dims=(0,), start_index_map=(0,))


def _bcast(v16, lane):
    idx = jnp.broadcast_to(lane, (L,)).astype(jnp.int32)[:, None]
    return lax.gather(v16, idx, _BCAST_DNUMS, (1,),
                      mode=lax.GatherScatterMode.PROMISE_IN_BOUNDS)


def _tc1_body(x_ref, w_ref, a_ref, h_ref, st_ref):
    h = x_ref[...] @ w_ref[...]
    h_ref[0] = h[:, :128]
    h_ref[1] = h[:, 128:]
    st_ref[...] = h @ a_ref[...]


def _tc1(x, Wcat, A):
    return pl.pallas_call(
        _tc1_body,
        grid=(GRID,),
        in_specs=[
            pl.BlockSpec((RB, IN_DIM), lambda i: (i, 0)),
            pl.BlockSpec((IN_DIM, NH * HID), lambda i: (0, 0)),
            pl.BlockSpec((NH * HID, L), lambda i: (0, 0)),
        ],
        out_specs=[
            pl.BlockSpec((NC, RB, 2 * HID), lambda i: (0, i, 0)),
            pl.BlockSpec((RB, L), lambda i: (i, 0)),
        ],
        out_shape=[
            jax.ShapeDtypeStruct((NC, N, 2 * HID), jnp.float32),
            jax.ShapeDtypeStruct((N, L), jnp.float32),
        ],
    )(x, Wcat, A)


def _sc1_body(src_hbm, dst_hbm, stf_hbm, h_hbm, zh_hbm, zr_hbm,
              rs_out, hp_out,
              src_v, dst_v, is0, is1, it0, it1, s0g, s1g, t0g, t1g,
              w0v, w1v, rows,
              src_vb, dst_vb, is0b, is1b, it0b, it1b, s0gb, s1gb, t0gb,
              t1gb, w0vb, w1vb, rowsb,
              rs0_sh, rs1_sh, st_sh, hp_sh,
              sem_r, sem_rb, sem_0, sem_1, sem_2, sem_3, sem_sa, sem_sb):
    c = lax.axis_index("c")
    s = lax.axis_index("s")
    c2 = 2 * c

    @pl.when(s == 1)
    def _():
        pltpu.sync_copy(stf_hbm, st_sh)

    @pl.when(s < NS - 1)
    def _():
        sl_hp = pl.ds(s * NR0, NR0)
        pltpu.sync_copy(zh_hbm.at[sl_hp], hp_sh.at[sl_hp])

    @pl.when(s == NS - 1)
    def _():
        sl_hp = pl.ds((NS - 1) * NR0, NR_LAST)
        pltpu.sync_copy(zh_hbm.at[sl_hp], hp_sh.at[sl_hp])

    @pl.when(s == 0)
    def _():
        pltpu.sync_copy(zr_hbm, rs0_sh)
        pltpu.sync_copy(zr_hbm, rs1_sh)

    plsc.subcore_barrier()

    per_tile = E // NS
    base0 = s * per_tile
    G = CH // L

    A = (src_v, dst_v, is0, is1, it0, it1, s0g, s1g, t0g, t1g, w0v, w1v,
         rows, sem_r)
    B = (src_vb, dst_vb, is0b, is1b, it0b, it1b, s0gb, s1gb, t0gb, t1gb,
         w0vb, w1vb, rowsb, sem_rb)

    def issue_chunk(base, bs):
        (bsrc, bdst, bis0, bis1, bit0, bit1, _, _, _, _, _, _, brows,
         bsem) = bs
        pltpu.sync_copy(src_hbm.at[pl.ds(base, CH)], bsrc)
        pltpu.sync_copy(dst_hbm.at[pl.ds(base, CH)], bdst)
        cp_r = pltpu.async_copy(h_hbm.at[c].at[bdst], brows, bsem)

        def idxcalc(g, carry2):
            sl = pl.ds(g * L, L)
            s16 = bsrc[sl] * L
            d16 = bdst[sl] * L
            bis0[sl] = s16 + c2
            bis1[sl] = s16 + (c2 + 1)
            bit0[sl] = d16 + (c2 + 4)
            bit1[sl] = d16 + (c2 + 5)
            return carry2

        lax.fori_loop(0, G, idxcalc, 0)
        return cp_r

    def st_issue(bs):
        (_, _, bis0, bis1, bit0, bit1, bs0g, bs1g, bt0g, bt1g, _, _, _,
         _) = bs
        return (pltpu.async_copy(st_sh.at[bis0], bs0g, sem_0),
                pltpu.async_copy(st_sh.at[bis1], bs1g, sem_1),
                pltpu.async_copy(st_sh.at[bit0], bt0g, sem_2),
                pltpu.async_copy(st_sh.at[bit1], bt1g, sem_3))

    def wphase(bs, cps):
        (bsrc, _, _, _, _, _, bs0g, bs1g, bt0g, bt1g, bw0, bw1, _, _) = bs
        for cp in cps:
            cp.wait()

        def wcalc(g, carry2):
            sl = pl.ds(g * L, L)
            e0 = bs0g[sl] + bt0g[sl]
            bw0[sl] = jnp.exp(-jnp.where(e0 >= 0, e0, ALPHA * e0))
            e1 = bs1g[sl] + bt1g[sl]
            bw1[sl] = jnp.exp(-jnp.where(e1 >= 0, e1, ALPHA * e1))
            return carry2

        lax.fori_loop(0, G, wcalc, 0)
        pltpu.sync_copy(bw0, rs0_sh.at[bsrc], add=True)
        pltpu.sync_copy(bw1, rs1_sh.at[bsrc], add=True)

    def finish(bs, cp_r, sem_s):
        (bsrc, _, _, _, _, _, _, _, _, _, bw0, bw1, brows, _) = bs
        cp_r.wait()

        def scale(g, carry2):
            kbase = g * L
            w0g = bw0[pl.ds(kbase, L)]
            w1g = bw1[pl.ds(kbase, L)]
            for j in range(L):
                k = kbase + j
                b0 = _bcast(w0g, j)
                b1 = _bcast(w1g, j)
                for f in range(4):
                    sl = pl.ds(f * L, L)
                    brows[k, sl] = brows[k, sl] * b0
                for f in range(4, 8):
                    sl = pl.ds(f * L, L)
                    brows[k, sl] = brows[k, sl] * b1
            return carry2

        lax.fori_loop(0, G, scale, 0)
        return pltpu.async_copy(brows, hp_sh.at[bsrc], sem_s, add=True)

    def pair(ci, carry):
        base_a = base0 + ci * (2 * CH)
        cp_ra = issue_chunk(base_a, A)
        st_a = st_issue(A)
        cp_rb = issue_chunk(base_a + CH, B)
        wphase(A, st_a)
        cp_sa = finish(A, cp_ra, sem_sa)
        st_b = st_issue(B)
        wphase(B, st_b)
        cp_sb = finish(B, cp_rb, sem_sb)
        cp_sa.wait()
        cp_sb.wait()
        return carry

    lax.fori_loop(0, per_tile // (2 * CH), pair, 0)

    plsc.subcore_barrier()

    @pl.when(s < NS - 1)
    def _():
        sl_hp = pl.ds(s * NR0, NR0)
        pltpu.sync_copy(hp_sh.at[sl_hp], hp_out.at[c].at[sl_hp])

    @pl.when(s == NS - 1)
    def _():
        sl_hp = pl.ds((NS - 1) * NR0, NR_LAST)
        pltpu.sync_copy(hp_sh.at[sl_hp], hp_out.at[c].at[sl_hp])

    @pl.when(s == 0)
    def _():
        pltpu.sync_copy(rs0_sh, rs_out.at[c2])
        pltpu.sync_copy(rs1_sh, rs_out.at[c2 + 1])


def _sc1(src, dst, STf, H, zh, zr1):
    mesh = plsc.VectorSubcoreMesh(core_axis_name="c", subcore_axis_name="s")
    kfn = pl.kernel(
        _sc1_body,
        out_type=[
            jax.ShapeDtypeStruct((NH, N), jnp.float32),
            jax.ShapeDtypeStruct((NC, N, 2 * HID), jnp.float32),
        ],
        mesh=mesh,
        scratch_types=(
            [pltpu.VMEM((CH,), jnp.int32)] * 6
            + [pltpu.VMEM((CH,), jnp.float32)] * 6
            + [pltpu.VMEM((CH, 2 * HID), jnp.float32)]
            + [pltpu.VMEM((CH,), jnp.int32)] * 6
            + [pltpu.VMEM((CH,), jnp.float32)] * 6
            + [pltpu.VMEM((CH, 2 * HID), jnp.float32)]
            + [
                pltpu.VMEM_SHARED((N,), jnp.float32),
                pltpu.VMEM_SHARED((N,), jnp.float32),
                pltpu.VMEM_SHARED((N * L,), jnp.float32),
                pltpu.VMEM_SHARED((N, 2 * HID), jnp.float32),
            ]
            + [pltpu.SemaphoreType.DMA] * 8
        ),
    )
    return kfn(src, dst, STf, H, zh, zr1)


def _tc2_body(hp_ref, r0_ref, r1_ref, r2_ref, r3_ref, wo_ref, alt_ref,
              h2_ref, s2_ref, t2_ref):
    rs = [r0_ref, r1_ref, r2_ref, r3_ref]
    cols = []
    for i in range(NH):
        hpc = hp_ref[i // 2][:, (i % 2) * HID:(i % 2 + 1) * HID]
        cols.append(_elu(hpc / (rs[i][...] + EPS)))
    x2 = jnp.concatenate(cols, axis=1)
    h2 = x2 @ wo_ref[...]
    h2_ref[...] = h2
    st = h2 @ alt_ref[...]
    s2_ref[...] = st[:, 0:1]
    t2_ref[...] = st[:, 1:2]


def _tc2(hp, r0, r1, r2, r3, W_out, ALT):
    return pl.pallas_call(
        _tc2_body,
        grid=(GRID,),
        in_specs=[
            pl.BlockSpec((NC, RB, 2 * HID), lambda i: (0, i, 0)),
            pl.BlockSpec((RB, 1), lambda i: (i, 0)),
            pl.BlockSpec((RB, 1), lambda i: (i, 0)),
            pl.BlockSpec((RB, 1), lambda i: (i, 0)),
            pl.BlockSpec((RB, 1), lambda i: (i, 0)),
            pl.BlockSpec((NH * HID, EMB), lambda i: (0, 0)),
            pl.BlockSpec((EMB, 2), lambda i: (0, 0)),
        ],
        out_specs=[
            pl.BlockSpec((RB, EMB), lambda i: (i, 0)),
            pl.BlockSpec((RB, 1), lambda i: (i, 0)),
            pl.BlockSpec((RB, 1), lambda i: (i, 0)),
        ],
        out_shape=[
            jax.ShapeDtypeStruct((N, EMB), jnp.float32),
            jax.ShapeDtypeStruct((N, 1), jnp.float32),
            jax.ShapeDtypeStruct((N, 1), jnp.float32),
        ],
    )(hp, r0, r1, r2, r3, W_out, ALT)


def _sc2_body(src_hbm, dst_hbm, h2_hbm, s2_hbm, t2_hbm, zh_hbm, zr_hbm,
              hp_out, rs_out, att_out,
              src_v, dst_v, sg, tg, wv, rows,
              src_vb, dst_vb, sgb, tgb, wvb, rowsb,
              hp_sh, rs_sh, s_sh, t_sh,
              sem_g0, sem_g1, sem_r, sem_rb, sem_sa, sem_sb):
    c = lax.axis_index("c")
    s = lax.axis_index("s")

    @pl.when(s < NS - 1)
    def _():
        sl_hp = pl.ds(s * NR0, NR0)
        pltpu.sync_copy(zh_hbm.at[sl_hp], hp_sh.at[sl_hp])

    @pl.when(s == NS - 1)
    def _():
        sl_hp = pl.ds((NS - 1) * NR0, NR_LAST)
        pltpu.sync_copy(zh_hbm.at[sl_hp], hp_sh.at[sl_hp])

    @pl.when(s == 0)
    def _():
        pltpu.sync_copy(zr_hbm, rs_sh)
        pltpu.sync_copy(s2_hbm, s_sh)
        pltpu.sync_copy(t2_hbm, t_sh)

    plsc.subcore_barrier()

    wid = s * NC + c
    per_tile = E // (NC * NS)
    base0 = wid * per_tile

    A = (src_v, dst_v, sg, tg, wv, rows, sem_r)
    B = (src_vb, dst_vb, sgb, tgb, wvb, rowsb, sem_rb)

    def issue2(base, bs):
        (bsrc, bdst, _, _, _, brows, bsem) = bs
        pltpu.sync_copy(src_hbm.at[pl.ds(base, CH)], bsrc)
        pltpu.sync_copy(dst_hbm.at[pl.ds(base, CH)], bdst)
        return pltpu.async_copy(h2_hbm.at[bdst], brows, bsem)

    def w2(base, bs):
        (bsrc, bdst, bsg, btg, bwv, _, _) = bs
        cs = pltpu.async_copy(s_sh.at[bsrc], bsg, sem_g0)
        ct = pltpu.async_copy(t_sh.at[bdst], btg, sem_g1)
        cs.wait()
        ct.wait()

        def wstep(i, carry2):
            sl = pl.ds(i * L, L)
            e = bsg[sl] + btg[sl]
            le = jnp.where(e >= 0, e, ALPHA * e)
            bwv[sl] = jnp.exp(-le)
            return carry2

        lax.fori_loop(0, CH // L, wstep, 0)
        pltpu.sync_copy(bwv, rs_sh.at[bsrc], add=True)
        pltpu.sync_copy(bwv, att_out.at[pl.ds(base, CH)])

    def fin2(bs, cp_r, sem_s):
        (bsrc, _, _, _, bwv, brows, _) = bs
        cp_r.wait()

        def scale(g, carry2):
            kbase = g * L
            wgrp = bwv[pl.ds(kbase, L)]
            for j in range(L):
                k = kbase + j
                w0 = _bcast(wgrp, j)
                for f in range(8):
                    sl = pl.ds(f * L, L)
                    brows[k, sl] = brows[k, sl] * w0
            return carry2

        lax.fori_loop(0, CH // L, scale, 0)
        return pltpu.async_copy(brows, hp_sh.at[bsrc], sem_s, add=True)

    NPAIR = per_tile // (2 * CH)

    def pair(ci, carry):
        base_a = base0 + ci * (2 * CH)
        cp_ra = issue2(base_a, A)
        cp_rb = issue2(base_a + CH, B)
        w2(base_a, A)
        cp_sa = fin2(A, cp_ra, sem_sa)
        w2(base_a + CH, B)
        cp_sb = fin2(B, cp_rb, sem_sb)
        cp_sa.wait()
        cp_sb.wait()
        return carry

    lax.fori_loop(0, NPAIR, pair, 0)

    base_t = base0 + NPAIR * (2 * CH)
    cp_rt = issue2(base_t, A)
    w2(base_t, A)
    fin2(A, cp_rt, sem_sa).wait()

    plsc.subcore_barrier()

    @pl.when(s < NS - 1)
    def _():
        sl_hp = pl.ds(s * NR0, NR0)
        pltpu.sync_copy(hp_sh.at[sl_hp], hp_out.at[c].at[sl_hp])

    @pl.when(s == NS - 1)
    def _():
        sl_hp = pl.ds((NS - 1) * NR0, NR_LAST)
        pltpu.sync_copy(hp_sh.at[sl_hp], hp_out.at[c].at[sl_hp])

    @pl.when(s == 0)
    def _():
        pltpu.sync_copy(rs_sh, rs_out.at[c])


def _sc2(src, dst, h2, s2, t2, zh, zr1):
    mesh = plsc.VectorSubcoreMesh(core_axis_name="c", subcore_axis_name="s")
    kfn = pl.kernel(
        _sc2_body,
        out_type=[
            jax.ShapeDtypeStruct((NC, N, EMB), jnp.float32),
            jax.ShapeDtypeStruct((NC, N), jnp.float32),
            jax.ShapeDtypeStruct((E,), jnp.float32),
        ],
        mesh=mesh,
        scratch_types=(
            [pltpu.VMEM((CH,), jnp.int32)] * 2
            + [pltpu.VMEM((CH,), jnp.float32)] * 3
            + [pltpu.VMEM((CH, EMB), jnp.float32)]
            + [pltpu.VMEM((CH,), jnp.int32)] * 2
            + [pltpu.VMEM((CH,), jnp.float32)] * 3
            + [pltpu.VMEM((CH, EMB), jnp.float32)]
            + [
                pltpu.VMEM_SHARED((N, EMB), jnp.float32),
                pltpu.VMEM_SHARED((N,), jnp.float32),
                pltpu.VMEM_SHARED((N,), jnp.float32),
                pltpu.VMEM_SHARED((N,), jnp.float32),
            ]
            + [pltpu.SemaphoreType.DMA] * 6
        ),
    )
    return kfn(src, dst, h2, s2, t2, zh, zr1)


def _tc3_body(hp_ref, rs_ref, out_ref):
    acc = hp_ref[0] + hp_ref[1]
    rsum = rs_ref[0] + rs_ref[1] + EPS
    out_ref[...] = _elu(acc / rsum)


def _tc3(hp2, rs2):
    return pl.pallas_call(
        _tc3_body,
        grid=(GRID,),
        in_specs=[
            pl.BlockSpec((NC, RB, EMB), lambda i: (0, i, 0)),
            pl.BlockSpec((NC, RB, 1), lambda i: (0, i, 0)),
        ],
        out_specs=pl.BlockSpec((RB, EMB), lambda i: (i, 0)),
        out_shape=jax.ShapeDtypeStruct((N, EMB), jnp.float32),
    )(hp2, rs2)


def kernel(adj, x, W0, a0, W1, a1, W2, a2, W3, a3, W_out, a_out):
    adj32 = adj.astype(jnp.int32)
    src = adj32[0]
    dst = adj32[1]

    Wcat = jnp.concatenate([W0, W1, W2, W3], axis=1)
    A = jnp.zeros((NH * HID, L), jnp.float32)
    for i, a in enumerate([a0, a1, a2, a3]):
        A = A.at[i * HID:(i + 1) * HID, i].set(a[0, :HID])
        A = A.at[i * HID:(i + 1) * HID, 4 + i].set(a[0, HID:])
    ALT = jnp.concatenate([a_out[:, :EMB].T, a_out[:, EMB:].T], axis=1)

    zh = jnp.zeros((N, 2 * HID), jnp.float32)
    zr1 = jnp.zeros((N,), jnp.float32)

    H, ST = _tc1(x, Wcat, A)
    rs, hp = _sc1(src, dst, ST.reshape(N * L), H, zh, zr1)
    h2, s2, t2 = _tc2(hp, rs[0].reshape(N, 1), rs[1].reshape(N, 1),
                      rs[2].reshape(N, 1), rs[3].reshape(N, 1), W_out, ALT)
    hp2, rs2, att = _sc2(src, dst, h2, s2.reshape(N), t2.reshape(N), zh, zr1)
    out = _tc3(hp2, rs2.reshape(NC, N, 1))
    return out, adj, att
